# baseline (speedup 1.0000x reference)
import jax
import jax.numpy as jnp
from jax.experimental import pallas as pl


def _gcn_conv(x, edge_index, W, b):
    n = x.shape[0]
    loops = jnp.arange(n, dtype=edge_index.dtype)
    src = jnp.concatenate([edge_index[0], loops])
    dst = jnp.concatenate([edge_index[1], loops])
    deg = jax.ops.segment_sum(jnp.ones(src.shape[0], dtype=x.dtype), dst, num_segments=n)
    dinv = jnp.where(deg > 0, jax.lax.rsqrt(jnp.maximum(deg, 1e-12)), 0.0)
    norm = dinv[src] * dinv[dst]
    xw = x @ W
    msg = xw[src] * norm[:, None]
    out = jax.ops.segment_sum(msg, dst, num_segments=n)
    return out + b


def kernel(x, edge_index, batch, W1, b1, W2, b2, RW1, Rb1, RW2, Rb2, RW3, Rb3, RW4, Rb4):
    G = 1000
    h = jax.nn.relu(_gcn_conv(x, edge_index, W1, b1))
    h = jax.nn.relu(_gcn_conv(h, edge_index, W2, b2))
    g = jax.ops.segment_sum(h, batch, num_segments=G)
    g = jax.nn.relu(g @ RW1 + Rb1)
    g = jax.nn.relu(g @ RW2 + Rb2)
    g = jax.nn.relu(g @ RW3 + Rb3)
    g = g @ RW4 + Rb4
    return g


# speedup vs baseline: 20.2561x; 20.2561x over previous
import functools

import jax
import jax.numpy as jnp
from jax import lax
from jax.experimental import pallas as pl
from jax.experimental.pallas import tpu as pltpu
from jax.experimental.pallas import tpu_sc as plsc

N = 100000
E = 1600000
G = 1000

NC = 2
NS = 16
LANES = 16

K_ACC = 102400
ZROWS = 256
KCH = 8

_mesh = plsc.VectorSubcoreMesh(core_axis_name="c", subcore_axis_name="s")


def _sc_pass(n_rows128, slices_per_core, k_acc, gather, rows_split_by_core):
    s_out = NC * slices_per_core
    if rows_split_by_core:
        rows_per_core = n_rows128 // NC
    else:
        rows_per_core = n_rows128
    rows_per_tile = rows_per_core // NS
    n_chunks = rows_per_tile // KCH
    assert rows_per_tile % KCH == 0
    stripe = k_acc // NS
    n_zcopy = (stripe + ZROWS - 1) // ZROWS
    assert stripe % ZROWS == 0 or stripe < ZROWS

    scratch = [
        pltpu.VMEM((KCH, 128), jnp.int32),
        pltpu.VMEM((KCH, 128, LANES), jnp.float32),
        pltpu.VMEM((min(ZROWS, stripe), LANES), jnp.float32),
        pltpu.VMEM_SHARED((k_acc, LANES), jnp.float32),
        pltpu.SemaphoreType.DMA,
    ]
    if gather:
        scratch.insert(0, pltpu.VMEM((KCH, 128), jnp.int32))

    out_type = jax.ShapeDtypeStruct((s_out, k_acc, LANES), jnp.float32)

    def body(*refs):
        if gather:
            table, gidx, dst, out, gbuf, dbuf, rbuf, zbuf, acc, sem = refs
        else:
            dst, out, dbuf, rbuf, zbuf, acc, sem = refs
        cid = lax.axis_index("c")
        sid = lax.axis_index("s")

        zn = min(ZROWS, stripe)
        @pl.loop(0, zn)
        def _(i):
            zbuf[i, :] = jnp.zeros((LANES,), jnp.float32)

        if not gather:
            @pl.loop(0, KCH * 128)
            def _(i):
                rbuf[i // 128, i % 128, :] = jnp.ones((LANES,), jnp.float32)

        for sl in range(slices_per_core):
            s = cid * slices_per_core + sl
            @pl.loop(0, n_zcopy)
            def _(i):
                pltpu.sync_copy(zbuf, acc.at[pl.ds(sid * stripe + i * zn, zn)])
            plsc.subcore_barrier()

            if rows_split_by_core:
                row0 = cid * rows_per_core + sid * rows_per_tile
            else:
                row0 = sid * rows_per_tile

            @pl.loop(0, n_chunks)
            def _(c):
                rbase = row0 + c * KCH
                pltpu.sync_copy(dst.at[pl.ds(rbase, KCH)], dbuf)
                if gather:
                    pltpu.sync_copy(gidx.at[s, pl.ds(rbase, KCH)], gbuf)
                    handles = [
                        pltpu.async_copy(table.at[gbuf.at[j]], rbuf.at[j], sem)
                        for j in range(KCH)
                    ]
                    for j in range(KCH):
                        handles[j].wait()
                        pltpu.sync_copy(rbuf.at[j], acc.at[dbuf.at[j]], add=True)
                else:
                    for j in range(KCH):
                        pltpu.sync_copy(rbuf.at[j], acc.at[dbuf.at[j]], add=True)
            plsc.subcore_barrier()

            @pl.loop(0, n_zcopy)
            def _(i):
                off = sid * stripe + i * zn
                pltpu.sync_copy(acc.at[pl.ds(off, zn)], out.at[s, pl.ds(off, zn)])
            plsc.subcore_barrier()

    return pl.kernel(
        body, out_type=out_type, mesh=_mesh, scratch_types=scratch,
        compiler_params=pltpu.CompilerParams(use_tc_tiling_on_sc=False),
    )


def _pad_idx(a, n_pad, pad_vals):
    return jnp.concatenate([a, pad_vals[:n_pad]])


def kernel(x, edge_index, batch, W1, b1, W2, b2, RW1, Rb1, RW2, Rb2, RW3, Rb3, RW4, Rb4):
    f32 = jnp.float32
    src = edge_index[0].astype(jnp.int32)
    dst = edge_index[1].astype(jnp.int32)
    batch32 = batch.astype(jnp.int32)

    rows_e = 12544
    e_pad = rows_e * 128
    n_pad_e = e_pad - E
    iot_e = lax.iota(jnp.int32, n_pad_e)
    srcp = _pad_idx(src, n_pad_e, iot_e % N)
    dstp = _pad_idx(dst, n_pad_e, N + (iot_e % 16))
    dst_rows = dstp.reshape(rows_e, 128)

    def gidx_for(stride):
        return (srcp[None, :] * stride
                + lax.iota(jnp.int32, stride)[:, None]).reshape(stride, rows_e, 128)

    deg_pass = _sc_pass(rows_e, 1, K_ACC, gather=False, rows_split_by_core=True)
    agg4_pass = _sc_pass(rows_e, 2, K_ACC, gather=True, rows_split_by_core=False)
    agg2_pass = _sc_pass(rows_e, 1, K_ACC, gather=True, rows_split_by_core=False)

    rows_p = 896
    p_pad = rows_p * 128
    n_pad_p = p_pad - N
    iot_p = lax.iota(jnp.int32, n_pad_p)
    psrc = _pad_idx(lax.iota(jnp.int32, N), n_pad_p, iot_p % N)
    pdst = _pad_idx(batch32, n_pad_p, (G + 8) + (iot_p % 16))
    pool_pass = _sc_pass(rows_p, 1, 1024, gather=True, rows_split_by_core=False)
    pgidx = (psrc[None, :] * 2
             + lax.iota(jnp.int32, 2)[:, None]).reshape(2, rows_p, 128)
    pdst_rows = pdst.reshape(rows_p, 128)

    deg_parts = deg_pass(dst_rows)
    deg = deg_parts[0, :N, 0] + deg_parts[1, :N, 0] + 1.0
    dinv = lax.rsqrt(deg)

    y1 = (x @ W1) * dinv[:, None]
    t1 = y1.reshape(N * 4, 16)
    agg1 = agg4_pass(t1, gidx_for(4), dst_rows)
    agg1 = agg1[:, :N, :].transpose(1, 0, 2).reshape(N, 64)
    h1 = jax.nn.relu(dinv[:, None] * (agg1 + y1) + b1)

    y2 = (h1 @ W2) * dinv[:, None]
    t2 = y2.reshape(N * 2, 16)
    agg2 = agg2_pass(t2, gidx_for(2), dst_rows)
    agg2 = agg2[:, :N, :].transpose(1, 0, 2).reshape(N, 32)
    h2 = jax.nn.relu(dinv[:, None] * (agg2 + y2) + b2)

    tp = h2.reshape(N * 2, 16)
    gp = pool_pass(tp, pgidx, pdst_rows)
    g = jnp.concatenate([gp[0, :G, :], gp[1, :G, :]], axis=1)

    g = jax.nn.relu(g @ RW1 + Rb1)
    g = jax.nn.relu(g @ RW2 + Rb2)
    g = jax.nn.relu(g @ RW3 + Rb3)
    g = g @ RW4 + Rb4
    return g
